# Initial kernel scaffold; baseline (speedup 1.0000x reference)
import jax
import jax.numpy as jnp
from jax.experimental import pallas as pl


def kernel(adjacency_matrix, graph, W, b):
    raise NotImplementedError("write your pallas kernel here")


# speedup vs baseline: 1.3841x; 1.3841x over previous
import functools

import jax
import jax.numpy as jnp
from jax import lax
from jax.experimental import pallas as pl
from jax.experimental.pallas import tpu as pltpu
from jax.experimental.pallas import tpu_sc as plsc

D = 128
K = 32
L = 16
NC, NS = 2, 16
NW = NC * NS
NB = 4
G = NB * K


def _make_gather_sum(n_pad):
  chunk = n_pad // NW
  nsub = chunk // NB
  mesh = plsc.VectorSubcoreMesh(core_axis_name="c", subcore_axis_name="s")

  @functools.partial(
      pl.kernel,
      mesh=mesh,
      out_type=jax.ShapeDtypeStruct((n_pad, D), jnp.float32),
      scratch_types=[
          pltpu.VMEM((nsub, G), jnp.int32),
          pltpu.VMEM((G, D), jnp.float32),
          pltpu.VMEM((chunk, D), jnp.float32),
          pltpu.SemaphoreType.DMA,
      ],
  )
  def gsum(h_hbm, idx_hbm, out_hbm, idx_v, gbuf, outv, sem):
    wid = lax.axis_index("c") * NS + lax.axis_index("s")
    pltpu.sync_copy(idx_hbm.at[wid], idx_v)

    def block(s, carry):
      pltpu.async_copy(h_hbm.at[idx_v.at[s]], gbuf, sem).wait()
      for n in range(NB):
        def kstep(k, acc, n=n):
          return tuple(acc[j] + gbuf[n * K + k, pl.ds(j * L, L)]
                       for j in range(D // L))
        acc = lax.fori_loop(
            1, K, kstep,
            tuple(gbuf[n * K, pl.ds(j * L, L)] for j in range(D // L)))
        for j in range(D // L):
          outv[s * NB + n, pl.ds(j * L, L)] = acc[j]
      return carry

    lax.fori_loop(0, nsub, block, 0)
    pltpu.sync_copy(outv, out_hbm.at[pl.ds(wid * chunk, chunk)])

  return gsum


def _make_mm_relu(n_pad, bm):
  def body(x_ref, w_ref, b_ref, o_ref):
    y = jnp.dot(x_ref[...], w_ref[...], preferred_element_type=jnp.float32)
    o_ref[...] = jnp.maximum(y * (1.0 / K) + b_ref[...], 0.0)

  return pl.pallas_call(
      body,
      grid=(n_pad // bm,),
      in_specs=[
          pl.BlockSpec((bm, D), lambda i: (i, 0)),
          pl.BlockSpec((D, D), lambda i: (0, 0)),
          pl.BlockSpec((1, D), lambda i: (0, 0)),
      ],
      out_specs=pl.BlockSpec((bm, D), lambda i: (i, 0)),
      out_shape=jax.ShapeDtypeStruct((n_pad, D), jnp.float32),
  )


def kernel(adjacency_matrix, graph, W, b):
  depth = W.shape[0]
  n = graph.shape[1]
  align = NW * NB * 8
  n_pad = ((n + align - 1) // align) * align

  h = jnp.pad(graph[0], ((0, n_pad - n), (0, 0)))
  idx = jnp.pad(adjacency_matrix.T.astype(jnp.int32),
                ((0, n_pad - n), (0, 0))).reshape(NW, -1, G)

  gsum = _make_gather_sum(n_pad)
  mm = _make_mm_relu(n_pad, 512)
  for t in range(1, depth):
    m = gsum(h, idx)
    h = mm(m, W[t], b[t].reshape(1, D))
  return h[:n][None]
